# baseline (speedup 1.0000x reference)
import functools

import jax
import jax.numpy as jnp
from jax import lax
from jax.experimental import pallas as pl
from jax.experimental.pallas import tpu as pltpu
from jax.experimental.pallas import tpu_sc as plsc

_EMB_DIM = 64
_FEAT_DIM = 128
_HID = 64
_BATCH = 16384

_NC = 2
_NS = 16
_NW = _NC * _NS
_B_PER_W = _BATCH // _NW
_DEPTH = 8


@functools.cache
def _make_sc_gather():
    @functools.partial(
        pl.kernel,
        mesh=plsc.VectorSubcoreMesh(core_axis_name="c", subcore_axis_name="s"),
        compiler_params=pltpu.CompilerParams(needs_layout_passes=False),
        out_type=jax.ShapeDtypeStruct((_BATCH * _EMB_DIM,), jnp.float32),
        scratch_types=[
            pltpu.VMEM((_B_PER_W + 16,), jnp.int32),
            pltpu.VMEM((_B_PER_W + 16,), jnp.int32),
            pltpu.VMEM((_DEPTH, _EMB_DIM, 128), jnp.float32),
            pltpu.VMEM((_B_PER_W * _EMB_DIM,), jnp.float32),
            pltpu.SemaphoreType.DMA((_DEPTH,)),
            pltpu.SemaphoreType.DMA,
        ],
    )
    def _sc_gather(tableT_hbm, w1_hbm, w2_hbm, out_hbm,
                   w1_v, w2_v, grp_v, rows_v, gsem, wsem):
        wid = lax.axis_index("s") * _NC + lax.axis_index("c")
        pltpu.sync_copy(w1_hbm.at[wid], w1_v)
        pltpu.sync_copy(w2_hbm.at[wid], w2_v)

        iota16 = lax.iota(jnp.int32, 16)

        def _fire(word1, slot):
            flag = (word1 >> 13) & 1

            @pl.when(flag == 1)
            def _():
                grp = word1 & 8191
                pltpu.async_copy(
                    tableT_hbm.at[pl.ds(0, _EMB_DIM),
                                  pl.ds(pl.multiple_of(grp * 128, 128), 128)],
                    grp_v.at[slot],
                    gsem.at[slot],
                )

        pre = w1_v[pl.ds(_B_PER_W, 16)]
        for t in range(_DEPTH - 1):
            _fire(pre[t], t)

        def _round(r, carry):
            vec1 = w1_v[pl.ds(r * _DEPTH, 16)]
            vec2 = w2_v[pl.ds(r * _DEPTH, 16)]
            for d in range(_DEPTH):
                k = r * _DEPTH + d
                w2c = vec2[d]
                slot = (w2c >> 7) & 7
                newgrp = (w2c >> 24) & 1

                @pl.when(newgrp == 1)
                def _():
                    pltpu.make_async_copy(
                        tableT_hbm.at[pl.ds(0, _EMB_DIM), pl.ds(0, 128)],
                        grp_v.at[slot],
                        gsem.at[slot],
                    ).wait()

                lane = jnp.broadcast_to(w2c & 127, (16,))
                slotv = jnp.broadcast_to(slot, (16,))
                dest = (w2c >> 10) & 16383
                for j in range(_EMB_DIM // 16):
                    col = plsc.load_gather(
                        grp_v, [slotv, iota16 + j * 16, lane]
                    )
                    rows_v[pl.ds(k * _EMB_DIM + j * 16, 16)] = col
                pltpu.async_copy(
                    rows_v.at[pl.ds(k * _EMB_DIM, _EMB_DIM)],
                    out_hbm.at[pl.ds(dest * _EMB_DIM, _EMB_DIM)],
                    wsem,
                )

                @pl.when(newgrp == 1)
                def _():
                    _fire(vec1[d], (slot + _DEPTH - 1) % _DEPTH)
            return carry

        lax.fori_loop(0, _B_PER_W // _DEPTH, _round, 0)

        def _drain(i, carry):
            pltpu.make_async_copy(
                rows_v.at[pl.ds(0, _EMB_DIM)],
                out_hbm.at[pl.ds(0, _EMB_DIM)],
                wsem,
            ).wait()
            return carry

        lax.fori_loop(0, _B_PER_W, _drain, 0, unroll=8)

    return _sc_gather


_BB = 1024


def _tc_mm_body(feat_ref, w_ref, b_ref, proj_ref):
    proj_ref[...] = lax.dot_general(
        feat_ref[...], w_ref[...],
        (((1,), (1,)), ((), ())),
        preferred_element_type=jnp.float32,
    ) + b_ref[...]


def _tc_cat_body(proj_ref, emb_ref, out_ref):
    out_ref[:, :_HID] = proj_ref[...]
    out_ref[:, _HID:] = emb_ref[...]


def _pack_words(ids32):
    iota = lax.iota(jnp.int32, _BATCH)
    sids, order = lax.sort([ids32, iota], num_keys=1)
    grp = (sids >> 7).reshape(_NW, _B_PER_W)
    lane = (sids & 127).reshape(_NW, _B_PER_W)
    dest = order.reshape(_NW, _B_PER_W)
    first = jnp.concatenate(
        [jnp.ones((_NW, 1), jnp.bool_), grp[:, 1:] != grp[:, :-1]], axis=1
    )
    jl = jnp.cumsum(first.astype(jnp.int32), axis=1) - 1
    slot = jl % _DEPTH

    w2 = lane | (slot << 7) | (dest << 10) | (first.astype(jnp.int32) << 24)

    pos2d = jnp.broadcast_to(
        lax.iota(jnp.int32, _B_PER_W)[None, :], (_NW, _B_PER_W)
    )
    ckey = jnp.where(first, pos2d, pos2d + _B_PER_W)
    _, fgrp_by_idx = lax.sort([ckey, grp], dimension=1, num_keys=1)
    nuniq = jl[:, -1:] + 1

    ahead = jl + (_DEPTH - 1)
    agrp = jnp.take_along_axis(
        fgrp_by_idx, jnp.minimum(ahead, _B_PER_W - 1), axis=1
    )
    w1 = jnp.where(first & (ahead < nuniq), agrp | (1 << 13), 0)
    tpos = lax.iota(jnp.int32, 16)[None, :]
    pre = jnp.where(tpos < jnp.minimum(nuniq, _DEPTH - 1),
                    fgrp_by_idx[:, :16] | (1 << 13), 0)
    w1 = jnp.concatenate([w1, pre], axis=1)
    w2 = jnp.concatenate([w2, jnp.zeros((_NW, 16), jnp.int32)], axis=1)
    return w1, w2


def kernel(feat, id, W, b, table):
    ids = id.astype(jnp.int32)
    w1, w2 = _pack_words(ids)
    emb = _make_sc_gather()(table.T, w1, w2).reshape(_BATCH, _EMB_DIM)
    cbb = 2048
    proj = pl.pallas_call(
        _tc_mm_body,
        grid=(_BATCH // cbb,),
        in_specs=[
            pl.BlockSpec((cbb, _FEAT_DIM), lambda i: (i, 0)),
            pl.BlockSpec((_HID, _FEAT_DIM), lambda i: (0, 0)),
            pl.BlockSpec((1, _HID), lambda i: (0, 0)),
        ],
        out_specs=pl.BlockSpec((cbb, _HID), lambda i: (i, 0)),
        out_shape=jax.ShapeDtypeStruct((_BATCH, _HID), jnp.float32),
    )(feat, W, b.reshape(1, _HID))
    out = pl.pallas_call(
        _tc_cat_body,
        grid=(_BATCH // cbb,),
        in_specs=[
            pl.BlockSpec((cbb, _HID), lambda i: (i, 0)),
            pl.BlockSpec((cbb, _EMB_DIM), lambda i: (i, 0)),
        ],
        out_specs=pl.BlockSpec((cbb, _HID + _EMB_DIM), lambda i: (i, 0)),
        out_shape=jax.ShapeDtypeStruct((_BATCH, _HID + _EMB_DIM), jnp.float32),
    )(proj, emb)
    return out


# speedup vs baseline: 1.0216x; 1.0216x over previous
import functools

import jax
import jax.numpy as jnp
from jax import lax
from jax.experimental import pallas as pl
from jax.experimental.pallas import tpu as pltpu
from jax.experimental.pallas import tpu_sc as plsc

_EMB_DIM = 64
_FEAT_DIM = 128
_HID = 64
_BATCH = 16384

_NC = 2
_NS = 16
_NW = _NC * _NS
_B_PER_W = _BATCH // _NW
_DEPTH = 8


@functools.cache
def _make_sc_gather():
    @functools.partial(
        pl.kernel,
        mesh=plsc.VectorSubcoreMesh(core_axis_name="c", subcore_axis_name="s"),
        compiler_params=pltpu.CompilerParams(needs_layout_passes=False),
        out_type=jax.ShapeDtypeStruct((_BATCH * _EMB_DIM,), jnp.float32),
        scratch_types=[
            pltpu.VMEM((_B_PER_W + 16,), jnp.int32),
            pltpu.VMEM((_B_PER_W + 16,), jnp.int32),
            pltpu.VMEM((_DEPTH, _EMB_DIM, 128), jnp.float32),
            pltpu.VMEM((_B_PER_W * _EMB_DIM,), jnp.float32),
            pltpu.SemaphoreType.DMA((_DEPTH,)),
            pltpu.SemaphoreType.DMA,
        ],
    )
    def _sc_gather(tableT_hbm, w1_hbm, w2_hbm, out_hbm,
                   w1_v, w2_v, grp_v, rows_v, gsem, wsem):
        wid = lax.axis_index("s") * _NC + lax.axis_index("c")
        pltpu.sync_copy(w1_hbm.at[wid], w1_v)
        pltpu.sync_copy(w2_hbm.at[wid], w2_v)

        iota16 = lax.iota(jnp.int32, 16)

        def _fire(word1, slot):
            flag = (word1 >> 13) & 1

            @pl.when(flag == 1)
            def _():
                grp = word1 & 8191
                pltpu.async_copy(
                    tableT_hbm.at[pl.ds(0, _EMB_DIM),
                                  pl.ds(pl.multiple_of(grp * 128, 128), 128)],
                    grp_v.at[slot],
                    gsem.at[slot],
                )

        pre = w1_v[pl.ds(_B_PER_W, 16)]
        for t in range(_DEPTH - 1):
            _fire(pre[t], t)

        def _round(r, carry):
            vec1 = w1_v[pl.ds(r * _DEPTH, 16)]
            vec2 = w2_v[pl.ds(r * _DEPTH, 16)]
            for d in range(_DEPTH):
                k = r * _DEPTH + d
                w2c = vec2[d]
                slot = (w2c >> 7) & 7
                newgrp = (w2c >> 24) & 1

                @pl.when(newgrp == 1)
                def _():
                    pltpu.make_async_copy(
                        tableT_hbm.at[pl.ds(0, _EMB_DIM), pl.ds(0, 128)],
                        grp_v.at[slot],
                        gsem.at[slot],
                    ).wait()

                lane = jnp.broadcast_to(w2c & 127, (16,))
                slotv = jnp.broadcast_to(slot, (16,))
                dest = (w2c >> 10) & 16383
                for j in range(_EMB_DIM // 16):
                    col = plsc.load_gather(
                        grp_v, [slotv, iota16 + j * 16, lane]
                    )
                    rows_v[pl.ds(k * _EMB_DIM + j * 16, 16)] = col
                pltpu.async_copy(
                    rows_v.at[pl.ds(k * _EMB_DIM, _EMB_DIM)],
                    out_hbm.at[pl.ds(dest * _EMB_DIM, _EMB_DIM)],
                    wsem,
                )

                @pl.when(newgrp == 1)
                def _():
                    _fire(vec1[d], (slot + _DEPTH - 1) % _DEPTH)
            return carry

        lax.fori_loop(0, _B_PER_W // _DEPTH, _round, 0)

        def _drain(i, carry):
            pltpu.make_async_copy(
                rows_v.at[pl.ds(0, _EMB_DIM)],
                out_hbm.at[pl.ds(0, _EMB_DIM)],
                wsem,
            ).wait()
            return carry

        lax.fori_loop(0, _B_PER_W, _drain, 0, unroll=8)

    return _sc_gather


_BB = 1024


def _tc_body(feat_ref, w_ref, b_ref, emb_ref, out_ref):
    proj = lax.dot_general(
        feat_ref[...], w_ref[...],
        (((1,), (1,)), ((), ())),
        preferred_element_type=jnp.float32,
    )
    out_ref[:, :_HID] = proj + b_ref[...]
    out_ref[:, _HID:] = emb_ref[...]


def _pack_words(ids32):
    iota = lax.iota(jnp.int32, _BATCH)
    sids, order = lax.sort([ids32, iota], num_keys=1)
    grp = (sids >> 7).reshape(_NW, _B_PER_W)
    lane = (sids & 127).reshape(_NW, _B_PER_W)
    dest = order.reshape(_NW, _B_PER_W)
    first = jnp.concatenate(
        [jnp.ones((_NW, 1), jnp.bool_), grp[:, 1:] != grp[:, :-1]], axis=1
    )
    jl = jnp.cumsum(first.astype(jnp.int32), axis=1) - 1
    slot = jl % _DEPTH

    w2 = lane | (slot << 7) | (dest << 10) | (first.astype(jnp.int32) << 24)

    pos2d = jnp.broadcast_to(
        lax.iota(jnp.int32, _B_PER_W)[None, :], (_NW, _B_PER_W)
    )
    ckey = jnp.where(first, pos2d, pos2d + _B_PER_W)
    _, fgrp_by_idx = lax.sort([ckey, grp], dimension=1, num_keys=1)
    nuniq = jl[:, -1:] + 1

    ahead = jl + (_DEPTH - 1)
    agrp = jnp.take_along_axis(
        fgrp_by_idx, jnp.minimum(ahead, _B_PER_W - 1), axis=1
    )
    w1 = jnp.where(first & (ahead < nuniq), agrp | (1 << 13), 0)
    tpos = lax.iota(jnp.int32, 16)[None, :]
    pre = jnp.where(tpos < jnp.minimum(nuniq, _DEPTH - 1),
                    fgrp_by_idx[:, :16] | (1 << 13), 0)
    w1 = jnp.concatenate([w1, pre], axis=1)
    w2 = jnp.concatenate([w2, jnp.zeros((_NW, 16), jnp.int32)], axis=1)
    return w1, w2


def kernel(feat, id, W, b, table):
    ids = id.astype(jnp.int32)
    w1, w2 = _pack_words(ids)
    emb = _make_sc_gather()(table.T, w1, w2).reshape(_BATCH, _EMB_DIM)
    cbb = 2048
    out = pl.pallas_call(
        _tc_body,
        grid=(_BATCH // cbb,),
        in_specs=[
            pl.BlockSpec((cbb, _FEAT_DIM), lambda i: (i, 0)),
            pl.BlockSpec((_HID, _FEAT_DIM), lambda i: (0, 0)),
            pl.BlockSpec((1, _HID), lambda i: (0, 0)),
            pl.BlockSpec((cbb, _EMB_DIM), lambda i: (i, 0)),
        ],
        out_specs=pl.BlockSpec((cbb, _HID + _EMB_DIM), lambda i: (i, 0)),
        out_shape=jax.ShapeDtypeStruct((_BATCH, _HID + _EMB_DIM), jnp.float32),
    )(feat, W, b.reshape(1, _HID), emb)
    return out
